# baseline (speedup 1.0000x reference)
import functools

import jax
import jax.numpy as jnp
from jax.experimental import pallas as pl
from jax.experimental.pallas import tpu as pltpu

HIGHEST = jax.lax.Precision.HIGHEST


def _router_kernel(x_ref, w_ref, b_ref, g_ref, counts_ref, psum_ref, nd_ref,
                   *, n_experts, capacity):
    xv = x_ref[...]
    T = xv.shape[0]
    logits = jnp.dot(xv.astype(jnp.bfloat16), w_ref[...].astype(jnp.bfloat16),
                     preferred_element_type=jnp.float32) + b_ref[...]
    mx = jnp.max(logits, axis=1, keepdims=True)
    el = jnp.exp(logits - mx)
    probs = el / jnp.sum(el, axis=1, keepdims=True)
    psum_ref[...] = jnp.sum(probs, axis=0, keepdims=True)

    eidx = jax.lax.broadcasted_iota(jnp.int32, (T, n_experts), 1)
    is_max = logits == mx
    route = jnp.min(jnp.where(is_max, eidx, n_experts), axis=1, keepdims=True)

    onehot = (eidx == route).astype(jnp.float32)
    counts_ref[...] = jnp.sum(onehot, axis=0, keepdims=True)

    r_io = jax.lax.broadcasted_iota(jnp.int32, (T, T), 0)
    c_io = jax.lax.broadcasted_iota(jnp.int32, (T, T), 1)
    tril = (r_io >= c_io).astype(jnp.float32)
    csum = jnp.dot(tril, onehot, precision=HIGHEST,
                   preferred_element_type=jnp.float32)
    pos = jnp.sum((csum - 1.0) * onehot, axis=1, keepdims=True)
    pos_i = jnp.round(pos).astype(jnp.int32)
    keep = pos_i < capacity
    g_ref[...] = jnp.where(keep, route * capacity + pos_i,
                           n_experts * capacity)
    nd_ref[...] = T - jnp.sum(keep.astype(jnp.int32), keepdims=True)


def _expert_kernel(g_ref, x_ref, w1_ref, b1_ref, w2_ref, b2_ref, y_ref,
                   buf_ref, *, capacity, n_ff_chunks):
    e = pl.program_id(0)
    f = pl.program_id(1)

    @pl.when(f == 0)
    def _():
        gv = g_ref[...]
        cidx = jax.lax.broadcasted_iota(jnp.int32, (capacity, gv.shape[1]), 0)
        m = (gv == e * capacity + cidx).astype(jnp.float32)
        buf_ref[...] = jnp.dot(m, x_ref[...], precision=HIGHEST,
                               preferred_element_type=jnp.float32)

    h = jnp.dot(buf_ref[...].astype(jnp.bfloat16),
                w1_ref[...].astype(jnp.bfloat16),
                preferred_element_type=jnp.float32) + b1_ref[...]
    h = 0.5 * h * (1.0 + jax.lax.erf(h * 0.7071067811865476))
    part = jnp.dot(h.astype(jnp.bfloat16), w2_ref[...].astype(jnp.bfloat16),
                   preferred_element_type=jnp.float32)

    @pl.when(f == 0)
    def _():
        y_ref[...] = part + b2_ref[...]

    @pl.when(f != 0)
    def _():
        y_ref[...] += part


def _combine_kernel(g_ref, x_ref, y_ref, out_ref, *, n_slots):
    gcol = g_ref[...]
    sidx = jax.lax.broadcasted_iota(jnp.int32, (gcol.shape[0], n_slots), 1)
    onehot = (gcol == sidx).astype(jnp.float32)
    gathered = jnp.dot(onehot, y_ref[...], precision=HIGHEST,
                       preferred_element_type=jnp.float32)
    keepf = (gcol < n_slots).astype(jnp.float32)
    out_ref[...] = gathered + (1.0 - keepf) * x_ref[...]


def kernel(x, Wsw, bsw, W1, b1, W2, b2):
    b, s, d = x.shape
    T = b * s
    E = Wsw.shape[1]
    FF = W1.shape[2]
    C = int(T * 1.25 / E)
    n_slots = E * C

    xt = x.reshape(T, d)

    g_col, counts, psum, nd = pl.pallas_call(
        functools.partial(_router_kernel, n_experts=E, capacity=C),
        out_shape=(
            jax.ShapeDtypeStruct((T, 1), jnp.int32),
            jax.ShapeDtypeStruct((1, E), jnp.float32),
            jax.ShapeDtypeStruct((1, E), jnp.float32),
            jax.ShapeDtypeStruct((1, 1), jnp.int32),
        ),
    )(xt, Wsw, bsw.reshape(1, E))

    g_row = g_col.reshape(1, T)

    n_ff_chunks = 2
    ffc = FF // n_ff_chunks
    y_flat = pl.pallas_call(
        functools.partial(_expert_kernel, capacity=C, n_ff_chunks=n_ff_chunks),
        grid=(E, n_ff_chunks),
        in_specs=[
            pl.BlockSpec((1, T), lambda e, f: (0, 0)),
            pl.BlockSpec((T, d), lambda e, f: (0, 0)),
            pl.BlockSpec((None, d, ffc), lambda e, f: (e, 0, f)),
            pl.BlockSpec((None, 1, ffc), lambda e, f: (e, 0, f)),
            pl.BlockSpec((None, ffc, d), lambda e, f: (e, f, 0)),
            pl.BlockSpec((None, 1, d), lambda e, f: (e, 0, 0)),
        ],
        out_specs=pl.BlockSpec((None, C, d), lambda e, f: (e, 0, 0)),
        out_shape=jax.ShapeDtypeStruct((E, C, d), jnp.float32),
        scratch_shapes=[pltpu.VMEM((C, d), jnp.float32)],
    )(g_row, xt, W1, b1.reshape(E, 1, FF), W2, b2.reshape(E, 1, d))

    y2 = y_flat.reshape(n_slots, d)

    tb = 256
    out_t = pl.pallas_call(
        functools.partial(_combine_kernel, n_slots=n_slots),
        grid=(T // tb,),
        in_specs=[
            pl.BlockSpec((tb, 1), lambda i: (i, 0)),
            pl.BlockSpec((tb, d), lambda i: (i, 0)),
            pl.BlockSpec((n_slots, d), lambda i: (0, 0)),
        ],
        out_specs=pl.BlockSpec((tb, d), lambda i: (i, 0)),
        out_shape=jax.ShapeDtypeStruct((T, d), jnp.float32),
    )(g_col, xt, y2)

    out = out_t.reshape(b, s, d)
    return (out, counts.reshape(E), psum.reshape(E), nd.reshape(()))


# speedup vs baseline: 1.5491x; 1.5491x over previous
import functools

import jax
import jax.numpy as jnp
from jax.experimental import pallas as pl
from jax.experimental.pallas import tpu as pltpu

HIGHEST = jax.lax.Precision.HIGHEST


def _router_kernel(x_ref, w_ref, b_ref, g_ref, counts_ref, psum_ref, nd_ref,
                   *, n_experts, capacity):
    xv = x_ref[...]
    T = xv.shape[0]
    logits = jnp.dot(xv.astype(jnp.bfloat16), w_ref[...].astype(jnp.bfloat16),
                     preferred_element_type=jnp.float32) + b_ref[...]
    mx = jnp.max(logits, axis=1, keepdims=True)
    el = jnp.exp(logits - mx)
    probs = el / jnp.sum(el, axis=1, keepdims=True)
    psum_ref[...] = jnp.sum(probs, axis=0, keepdims=True)

    eidx = jax.lax.broadcasted_iota(jnp.int32, (T, n_experts), 1)
    is_max = logits == mx
    route = jnp.min(jnp.where(is_max, eidx, n_experts), axis=1, keepdims=True)

    onehot = (eidx == route).astype(jnp.float32)
    counts_ref[...] = jnp.sum(onehot, axis=0, keepdims=True)

    r_io = jax.lax.broadcasted_iota(jnp.int32, (T, T), 0)
    c_io = jax.lax.broadcasted_iota(jnp.int32, (T, T), 1)
    tril = (r_io >= c_io).astype(jnp.bfloat16)
    csum = jnp.dot(tril, onehot.astype(jnp.bfloat16),
                   preferred_element_type=jnp.float32)
    pos = jnp.sum((csum - 1.0) * onehot, axis=1, keepdims=True)
    pos_i = jnp.round(pos).astype(jnp.int32)
    keep = pos_i < capacity
    g_ref[...] = jnp.where(keep, route * capacity + pos_i,
                           n_experts * capacity)
    nd_ref[...] = T - jnp.sum(keep.astype(jnp.int32), keepdims=True)


def _expert_kernel(g_ref, x_ref, w1_ref, b1_ref, w2_ref, b2_ref, y_ref,
                   buf_ref, *, capacity, n_ff_chunks):
    e = pl.program_id(0)
    f = pl.program_id(1)

    @pl.when(f == 0)
    def _():
        gv = g_ref[...]
        cidx = jax.lax.broadcasted_iota(jnp.int32, (capacity, gv.shape[1]), 0)
        m = (gv == e * capacity + cidx).astype(jnp.bfloat16)
        buf_ref[...] = jnp.dot(m, x_ref[...].astype(jnp.bfloat16),
                               preferred_element_type=jnp.float32)

    h = jnp.dot(buf_ref[...].astype(jnp.bfloat16),
                w1_ref[...].astype(jnp.bfloat16),
                preferred_element_type=jnp.float32) + b1_ref[...]
    h = 0.5 * h * (1.0 + jax.lax.erf(h * 0.7071067811865476))
    part = jnp.dot(h.astype(jnp.bfloat16), w2_ref[...].astype(jnp.bfloat16),
                   preferred_element_type=jnp.float32)

    @pl.when(f == 0)
    def _():
        y_ref[...] = part + b2_ref[...]

    @pl.when(f != 0)
    def _():
        y_ref[...] += part


def _combine_kernel(g_ref, x_ref, y_ref, out_ref, *, n_slots):
    gcol = g_ref[...]
    sidx = jax.lax.broadcasted_iota(jnp.int32, (gcol.shape[0], n_slots), 1)
    onehot = (gcol == sidx).astype(jnp.bfloat16)
    gathered = jnp.dot(onehot, y_ref[...].astype(jnp.bfloat16),
                       preferred_element_type=jnp.float32)
    keepf = (gcol < n_slots).astype(jnp.float32)
    out_ref[...] = gathered + (1.0 - keepf) * x_ref[...]


def kernel(x, Wsw, bsw, W1, b1, W2, b2):
    b, s, d = x.shape
    T = b * s
    E = Wsw.shape[1]
    FF = W1.shape[2]
    C = int(T * 1.25 / E)
    n_slots = E * C

    xt = x.reshape(T, d)

    g_col, counts, psum, nd = pl.pallas_call(
        functools.partial(_router_kernel, n_experts=E, capacity=C),
        out_shape=(
            jax.ShapeDtypeStruct((T, 1), jnp.int32),
            jax.ShapeDtypeStruct((1, E), jnp.float32),
            jax.ShapeDtypeStruct((1, E), jnp.float32),
            jax.ShapeDtypeStruct((1, 1), jnp.int32),
        ),
    )(xt, Wsw, bsw.reshape(1, E))

    g_row = g_col.reshape(1, T)

    n_ff_chunks = 2
    ffc = FF // n_ff_chunks
    y_flat = pl.pallas_call(
        functools.partial(_expert_kernel, capacity=C, n_ff_chunks=n_ff_chunks),
        grid=(E, n_ff_chunks),
        in_specs=[
            pl.BlockSpec((1, T), lambda e, f: (0, 0)),
            pl.BlockSpec((T, d), lambda e, f: (0, 0)),
            pl.BlockSpec((None, d, ffc), lambda e, f: (e, 0, f)),
            pl.BlockSpec((None, 1, ffc), lambda e, f: (e, 0, f)),
            pl.BlockSpec((None, ffc, d), lambda e, f: (e, f, 0)),
            pl.BlockSpec((None, 1, d), lambda e, f: (e, 0, 0)),
        ],
        out_specs=pl.BlockSpec((None, C, d), lambda e, f: (e, 0, 0)),
        out_shape=jax.ShapeDtypeStruct((E, C, d), jnp.float32),
        scratch_shapes=[pltpu.VMEM((C, d), jnp.float32)],
    )(g_row, xt, W1, b1.reshape(E, 1, FF), W2, b2.reshape(E, 1, d))

    y2 = y_flat.reshape(n_slots, d)

    tb = 256
    out_t = pl.pallas_call(
        functools.partial(_combine_kernel, n_slots=n_slots),
        grid=(T // tb,),
        in_specs=[
            pl.BlockSpec((tb, 1), lambda i: (i, 0)),
            pl.BlockSpec((tb, d), lambda i: (i, 0)),
            pl.BlockSpec((n_slots, d), lambda i: (0, 0)),
        ],
        out_specs=pl.BlockSpec((tb, d), lambda i: (i, 0)),
        out_shape=jax.ShapeDtypeStruct((T, d), jnp.float32),
    )(g_col, xt, y2)

    out = out_t.reshape(b, s, d)
    return (out, counts.reshape(E), psum.reshape(E), nd.reshape(()))


# speedup vs baseline: 1.5540x; 1.0032x over previous
import functools

import jax
import jax.numpy as jnp
from jax.experimental import pallas as pl
from jax.experimental.pallas import tpu as pltpu

HIGHEST = jax.lax.Precision.HIGHEST


def _router_kernel(x_ref, w_ref, b_ref, g_ref, counts_ref, psum_ref, nd_ref,
                   *, n_experts, capacity):
    xv = x_ref[...]
    T = xv.shape[0]
    logits = jnp.dot(xv.astype(jnp.bfloat16), w_ref[...].astype(jnp.bfloat16),
                     preferred_element_type=jnp.float32) + b_ref[...]
    mx = jnp.max(logits, axis=1, keepdims=True)
    el = jnp.exp(logits - mx)
    probs = el / jnp.sum(el, axis=1, keepdims=True)
    psum_ref[...] = jnp.sum(probs, axis=0, keepdims=True)

    eidx = jax.lax.broadcasted_iota(jnp.int32, (T, n_experts), 1)
    is_max = logits == mx
    route = jnp.min(jnp.where(is_max, eidx, n_experts), axis=1, keepdims=True)

    onehot = (eidx == route).astype(jnp.float32)
    counts_ref[...] = jnp.sum(onehot, axis=0, keepdims=True)

    r_io = jax.lax.broadcasted_iota(jnp.int32, (T, T), 0)
    c_io = jax.lax.broadcasted_iota(jnp.int32, (T, T), 1)
    tril = (r_io >= c_io).astype(jnp.bfloat16)
    csum = jnp.dot(tril, onehot.astype(jnp.bfloat16),
                   preferred_element_type=jnp.float32)
    pos = jnp.sum((csum - 1.0) * onehot, axis=1, keepdims=True)
    pos_i = jnp.round(pos).astype(jnp.int32)
    keep = pos_i < capacity
    g_ref[...] = jnp.where(keep, route * capacity + pos_i,
                           n_experts * capacity)
    nd_ref[...] = T - jnp.sum(keep.astype(jnp.int32), keepdims=True)


def _expert_kernel(g_ref, x_ref, w1_ref, b1_ref, w2_ref, b2_ref, y_ref,
                   buf_ref, *, capacity, n_ff_chunks):
    e = pl.program_id(0)
    f = pl.program_id(1)

    @pl.when(f == 0)
    def _():
        gv = g_ref[...]
        cidx = jax.lax.broadcasted_iota(jnp.int32, (capacity, gv.shape[1]), 0)
        m = (gv == e * capacity + cidx).astype(jnp.bfloat16)
        buf_ref[...] = jnp.dot(m, x_ref[...],
                               preferred_element_type=jnp.float32)

    h = jnp.dot(buf_ref[...], w1_ref[...],
                preferred_element_type=jnp.float32) + b1_ref[...]
    h = 0.5 * h * (1.0 + jax.lax.erf(h * 0.7071067811865476))
    part = jnp.dot(h, w2_ref[...], preferred_element_type=jnp.float32)

    @pl.when(f == 0)
    def _():
        y_ref[...] = part + b2_ref[...]

    @pl.when(f != 0)
    def _():
        y_ref[...] += part


def _combine_kernel(g_ref, x_ref, y_ref, out_ref, *, n_slots):
    gcol = g_ref[...]
    sidx = jax.lax.broadcasted_iota(jnp.int32, (gcol.shape[0], n_slots), 1)
    onehot = (gcol == sidx).astype(jnp.bfloat16)
    gathered = jnp.dot(onehot, y_ref[...].astype(jnp.bfloat16),
                       preferred_element_type=jnp.float32)
    keepf = (gcol < n_slots).astype(jnp.float32)
    out_ref[...] = gathered + (1.0 - keepf) * x_ref[...]


def kernel(x, Wsw, bsw, W1, b1, W2, b2):
    b, s, d = x.shape
    T = b * s
    E = Wsw.shape[1]
    FF = W1.shape[2]
    C = int(T * 1.25 / E)
    n_slots = E * C

    xt = x.reshape(T, d)

    g_col, counts, psum, nd = pl.pallas_call(
        functools.partial(_router_kernel, n_experts=E, capacity=C),
        out_shape=(
            jax.ShapeDtypeStruct((T, 1), jnp.int32),
            jax.ShapeDtypeStruct((1, E), jnp.float32),
            jax.ShapeDtypeStruct((1, E), jnp.float32),
            jax.ShapeDtypeStruct((1, 1), jnp.int32),
        ),
    )(xt, Wsw, bsw.reshape(1, E))

    g_row = g_col.reshape(1, T)
    xtb = xt.astype(jnp.bfloat16)

    n_ff_chunks = 2
    ffc = FF // n_ff_chunks
    y_flat = pl.pallas_call(
        functools.partial(_expert_kernel, capacity=C, n_ff_chunks=n_ff_chunks),
        grid=(E, n_ff_chunks),
        in_specs=[
            pl.BlockSpec((1, T), lambda e, f: (0, 0)),
            pl.BlockSpec((T, d), lambda e, f: (0, 0)),
            pl.BlockSpec((None, d, ffc), lambda e, f: (e, 0, f)),
            pl.BlockSpec((None, 1, ffc), lambda e, f: (e, 0, f)),
            pl.BlockSpec((None, ffc, d), lambda e, f: (e, f, 0)),
            pl.BlockSpec((None, 1, d), lambda e, f: (e, 0, 0)),
        ],
        out_specs=pl.BlockSpec((None, C, d), lambda e, f: (e, 0, 0)),
        out_shape=jax.ShapeDtypeStruct((E, C, d), jnp.float32),
        scratch_shapes=[pltpu.VMEM((C, d), jnp.float32)],
    )(g_row, xtb, W1, b1.reshape(E, 1, FF), W2, b2.reshape(E, 1, d))

    y2 = y_flat.reshape(n_slots, d)

    tb = 256
    out_t = pl.pallas_call(
        functools.partial(_combine_kernel, n_slots=n_slots),
        grid=(T // tb,),
        in_specs=[
            pl.BlockSpec((tb, 1), lambda i: (i, 0)),
            pl.BlockSpec((tb, d), lambda i: (i, 0)),
            pl.BlockSpec((n_slots, d), lambda i: (0, 0)),
        ],
        out_specs=pl.BlockSpec((tb, d), lambda i: (i, 0)),
        out_shape=jax.ShapeDtypeStruct((T, d), jnp.float32),
    )(g_col, xt, y2)

    out = out_t.reshape(b, s, d)
    return (out, counts.reshape(E), psum.reshape(E), nd.reshape(()))


# speedup vs baseline: 1.5607x; 1.0043x over previous
import functools

import jax
import jax.numpy as jnp
from jax.experimental import pallas as pl
from jax.experimental.pallas import tpu as pltpu

HIGHEST = jax.lax.Precision.HIGHEST


def _router_kernel(x_ref, w_ref, b_ref, g_ref, counts_ref, psum_ref, nd_ref,
                   *, n_experts, capacity):
    xv = x_ref[...]
    T = xv.shape[0]
    logits = jnp.dot(xv.astype(jnp.bfloat16), w_ref[...].astype(jnp.bfloat16),
                     preferred_element_type=jnp.float32) + b_ref[...]
    mx = jnp.max(logits, axis=1, keepdims=True)
    el = jnp.exp(logits - mx)
    probs = el / jnp.sum(el, axis=1, keepdims=True)
    psum_ref[...] = jnp.sum(probs, axis=0, keepdims=True)

    eidx = jax.lax.broadcasted_iota(jnp.int32, (T, n_experts), 1)
    is_max = logits == mx
    route = jnp.min(jnp.where(is_max, eidx, n_experts), axis=1, keepdims=True)

    onehot = (eidx == route).astype(jnp.float32)
    counts_ref[...] = jnp.sum(onehot, axis=0, keepdims=True)

    r_io = jax.lax.broadcasted_iota(jnp.int32, (T, T), 0)
    c_io = jax.lax.broadcasted_iota(jnp.int32, (T, T), 1)
    tril = (r_io >= c_io).astype(jnp.bfloat16)
    csum = jnp.dot(tril, onehot.astype(jnp.bfloat16),
                   preferred_element_type=jnp.float32)
    pos = jnp.sum((csum - 1.0) * onehot, axis=1, keepdims=True)
    pos_i = jnp.round(pos).astype(jnp.int32)
    keep = pos_i < capacity
    g_ref[...] = jnp.where(keep, route * capacity + pos_i,
                           n_experts * capacity)
    nd_ref[...] = T - jnp.sum(keep.astype(jnp.int32), keepdims=True)


def _expert_kernel(g_ref, x_ref, w1_ref, b1_ref, w2_ref, b2_ref, y_ref,
                   buf_ref, *, capacity, n_ff_chunks):
    e = pl.program_id(0)
    f = pl.program_id(1)

    @pl.when(f == 0)
    def _():
        gv = g_ref[...]
        cidx = jax.lax.broadcasted_iota(jnp.int32, (capacity, gv.shape[1]), 0)
        m = (gv == e * capacity + cidx).astype(jnp.bfloat16)
        buf_ref[...] = jnp.dot(m, x_ref[...],
                               preferred_element_type=jnp.float32)

    h = jnp.dot(buf_ref[...], w1_ref[...],
                preferred_element_type=jnp.float32) + b1_ref[...]
    h = 0.5 * h * (1.0 + jax.lax.erf(h * 0.7071067811865476))
    part = jnp.dot(h, w2_ref[...], preferred_element_type=jnp.float32)

    @pl.when(f == 0)
    def _():
        y_ref[...] = part + b2_ref[...]

    @pl.when(f != 0)
    def _():
        y_ref[...] += part


def _combine_kernel(g_ref, x_ref, y_ref, out_ref, *, n_slots):
    gcol = g_ref[...]
    sidx = jax.lax.broadcasted_iota(jnp.int32, (gcol.shape[0], n_slots), 1)
    onehot = (gcol == sidx).astype(jnp.bfloat16)
    gathered = jnp.dot(onehot, y_ref[...].astype(jnp.bfloat16),
                       preferred_element_type=jnp.float32)
    keepf = (gcol < n_slots).astype(jnp.float32)
    out_ref[...] = gathered + (1.0 - keepf) * x_ref[...]


def kernel(x, Wsw, bsw, W1, b1, W2, b2):
    b, s, d = x.shape
    T = b * s
    E = Wsw.shape[1]
    FF = W1.shape[2]
    C = int(T * 1.25 / E)
    n_slots = E * C

    xt = x.reshape(T, d)

    g_col, counts, psum, nd = pl.pallas_call(
        functools.partial(_router_kernel, n_experts=E, capacity=C),
        out_shape=(
            jax.ShapeDtypeStruct((T, 1), jnp.int32),
            jax.ShapeDtypeStruct((1, E), jnp.float32),
            jax.ShapeDtypeStruct((1, E), jnp.float32),
            jax.ShapeDtypeStruct((1, 1), jnp.int32),
        ),
    )(xt, Wsw, bsw.reshape(1, E))

    g_row = g_col.reshape(1, T)
    xtb = xt.astype(jnp.bfloat16)

    n_ff_chunks = 1
    ffc = FF // n_ff_chunks
    y_flat = pl.pallas_call(
        functools.partial(_expert_kernel, capacity=C, n_ff_chunks=n_ff_chunks),
        grid=(E, n_ff_chunks),
        in_specs=[
            pl.BlockSpec((1, T), lambda e, f: (0, 0)),
            pl.BlockSpec((T, d), lambda e, f: (0, 0)),
            pl.BlockSpec((None, d, ffc), lambda e, f: (e, 0, f)),
            pl.BlockSpec((None, 1, ffc), lambda e, f: (e, 0, f)),
            pl.BlockSpec((None, ffc, d), lambda e, f: (e, f, 0)),
            pl.BlockSpec((None, 1, d), lambda e, f: (e, 0, 0)),
        ],
        out_specs=pl.BlockSpec((None, C, d), lambda e, f: (e, 0, 0)),
        out_shape=jax.ShapeDtypeStruct((E, C, d), jnp.float32),
        scratch_shapes=[pltpu.VMEM((C, d), jnp.float32)],
    )(g_row, xtb, W1, b1.reshape(E, 1, FF), W2, b2.reshape(E, 1, d))

    y2 = y_flat.reshape(n_slots, d)

    tb = 256
    out_t = pl.pallas_call(
        functools.partial(_combine_kernel, n_slots=n_slots),
        grid=(T // tb,),
        in_specs=[
            pl.BlockSpec((tb, 1), lambda i: (i, 0)),
            pl.BlockSpec((tb, d), lambda i: (i, 0)),
            pl.BlockSpec((n_slots, d), lambda i: (0, 0)),
        ],
        out_specs=pl.BlockSpec((tb, d), lambda i: (i, 0)),
        out_shape=jax.ShapeDtypeStruct((T, d), jnp.float32),
    )(g_col, xt, y2)

    out = out_t.reshape(b, s, d)
    return (out, counts.reshape(E), psum.reshape(E), nd.reshape(()))
